# baseline (speedup 1.0000x reference)
import functools
import math

import jax
import jax.numpy as jnp
from jax import lax
from jax.experimental import pallas as pl
from jax.experimental.pallas import tpu as pltpu
from jax.experimental.pallas import tpu_sc as plsc

EMBED_DIM = 32
SCALE = math.sqrt(EMBED_DIM)

NUM_CORES = 2
NUM_SUBCORES = 16
NUM_WORKERS = NUM_CORES * NUM_SUBCORES

RW = 128
NWIN = 4
C0 = 5
NCHUNK = 10
TBUF = 4


@functools.lru_cache(maxsize=None)
def _build_prep(num_rows: int, num_cols: int, vocab: int):
    n_tiles_c = (num_cols + 7) // 8
    win_per_worker = num_rows // RW // NUM_WORKERS
    n_full = vocab // RW
    vocab_pad = (n_full + 1) * RW
    per_w = n_full // NUM_WORKERS
    n_extra = n_full - per_w * NUM_WORKERS
    n_super = per_w // TBUF
    assert per_w % TBUF == 0
    mesh = plsc.VectorSubcoreMesh(core_axis_name="c", subcore_axis_name="s")

    @functools.partial(
        pl.kernel,
        mesh=mesh,
        out_type=(
            jax.ShapeDtypeStruct((num_rows * num_cols,), jnp.int32),
            jax.ShapeDtypeStruct((vocab_pad * EMBED_DIM // RW, RW), jnp.float32),
        ),
        scratch_types=[
            pltpu.VMEM((n_tiles_c * 8, RW), jnp.int32),
            pltpu.VMEM((TBUF, EMBED_DIM, RW), jnp.float32),
            pltpu.VMEM((TBUF, EMBED_DIM, RW), jnp.float32),
            pltpu.SemaphoreType.DMA,
            pltpu.SemaphoreType.DMA,
            pltpu.SemaphoreType.DMA((TBUF,)),
            pltpu.SemaphoreType.DMA((TBUF,)),
        ],
        compiler_params=pltpu.CompilerParams(
            use_tc_tiling_on_sc=True, needs_layout_passes=False
        ),
    )
    def prep(xt, tt, tailp, idx_out, tbl_out, stag, tstag, osb,
             isem, osem, tsem, wsem):
        wid = lax.axis_index("s") * NUM_CORES + lax.axis_index("c")
        iota16 = lax.iota(jnp.int32, 16)
        rowvs = [
            lax.shift_right_logical(iota16, 2) + i16 * 4 for i16 in range(8)
        ]
        colbase = lax.bitwise_and(iota16, 3) * EMBED_DIM

        def fire_tin(j, b):
            i0 = j * RW
            for qe in range(EMBED_DIM // 8):
                pltpu.async_copy(
                    tt.at[pl.ds(qe * 8, 8), pl.ds(i0, RW)],
                    tstag.at[b, pl.ds(qe * 8, 8)],
                    tsem.at[b],
                )

        def wait_tin(b):
            pltpu.make_async_copy(
                tt.at[pl.ds(0, EMBED_DIM), pl.ds(0, RW)], tstag.at[b], tsem.at[b]
            ).wait()

        def tblock(b):
            for e in range(EMBED_DIM):
                colv = colbase + e
                for i16 in range(8):
                    v = tstag[b, e, pl.ds(i16 * 16, 16)] * SCALE
                    plsc.store_scatter(osb.at[b], [rowvs[i16], colv], v)

        def fire_tout(j, b):
            pltpu.async_copy(
                osb.at[b], tbl_out.at[pl.ds(j * EMBED_DIM, EMBED_DIM)], wsem.at[b]
            )

        def wait_tout(b):
            pltpu.make_async_copy(
                tbl_out.at[pl.ds(0, EMBED_DIM)], osb.at[b], wsem.at[b]
            ).wait()

        j0 = wid * per_w
        for l in range(TBUF - 1):
            fire_tin(j0 + l, l)

        def super_body(s, carry):
            for b2 in range(TBUF):
                l = s * TBUF + b2
                j = j0 + l

                @pl.when(l + TBUF - 1 < per_w)
                def _():
                    fire_tin(j + TBUF - 1, (b2 + TBUF - 1) % TBUF)

                wait_tin(b2)

                @pl.when(l >= TBUF)
                def _():
                    wait_tout(b2)

                tblock(b2)
                fire_tout(j, b2)
            return carry

        lax.fori_loop(0, n_super, super_body, 0)
        for b2 in range(TBUF):
            wait_tout(b2)

        def oneblock(j, src_is_tail):
            if src_is_tail:
                for qe in range(EMBED_DIM // 8):
                    pltpu.async_copy(
                        tailp.at[pl.ds(qe * 8, 8)],
                        tstag.at[0, pl.ds(qe * 8, 8)],
                        tsem.at[0],
                    )
            else:
                fire_tin(j, 0)
            wait_tin(0)
            tblock(0)
            fire_tout(j, 0)
            wait_tout(0)

        if n_extra:
            @pl.when(wid < n_extra)
            def _():
                oneblock(NUM_WORKERS * per_w + wid, False)

        @pl.when(wid == NUM_WORKERS - 1)
        def _():
            oneblock(n_full, True)

        def win(k, carry):
            wdg = wid * win_per_worker + k
            r0 = wdg * RW
            for q in range(n_tiles_c):
                h = min(8, num_cols - q * 8)
                pltpu.async_copy(
                    xt.at[pl.ds(q * 8, h), pl.ds(r0, RW)],
                    stag.at[pl.ds(q * 8, h)],
                    isem,
                )
            for q in range(n_tiles_c):
                h = min(8, num_cols - q * 8)
                pltpu.make_async_copy(
                    xt.at[pl.ds(q * 8, h), pl.ds(r0, RW)],
                    stag.at[pl.ds(q * 8, h)],
                    isem,
                ).wait()
            for c in range(num_cols):
                pltpu.async_copy(
                    stag.at[c],
                    idx_out.at[pl.ds((wdg * num_cols + c) * RW, RW)],
                    osem,
                )
            for c in range(num_cols):
                pltpu.make_async_copy(
                    stag.at[c],
                    idx_out.at[pl.ds((wdg * num_cols + c) * RW, RW)],
                    osem,
                ).wait()
            return carry

        lax.fori_loop(0, win_per_worker, win, 0)

    return prep


@functools.lru_cache(maxsize=None)
def _build_lookup(num_rows: int, num_cols: int, vocab_pad: int):
    assert num_rows == NUM_WORKERS * NWIN * RW
    assert num_cols == C0 * NCHUNK
    slab = num_cols * RW
    mesh = plsc.VectorSubcoreMesh(core_axis_name="c", subcore_axis_name="s")

    @functools.partial(
        pl.kernel,
        mesh=mesh,
        out_type=jax.ShapeDtypeStruct(
            (num_cols, EMBED_DIM, num_rows), jnp.float32
        ),
        scratch_types=[
            pltpu.VMEM((slab,), jnp.int32),
            pltpu.VMEM((2, C0 * RW, EMBED_DIM), jnp.float32),
            pltpu.VMEM((C0, EMBED_DIM, RW), jnp.float32),
            pltpu.SemaphoreType.DMA,
            pltpu.SemaphoreType.DMA((2,)),
            pltpu.SemaphoreType.DMA,
        ],
        compiler_params=pltpu.CompilerParams(
            use_tc_tiling_on_sc=False, needs_layout_passes=False
        ),
    )
    def emb(idx_hbm, table_hbm, out_hbm, idx_v, rows_v, obuf_v, isem, gsem, osem):
        wid = lax.axis_index("s") * NUM_CORES + lax.axis_index("c")
        iota16 = lax.iota(jnp.int32, 16)

        def fire_gathers(cc, b):
            for c in range(C0):
                pltpu.async_copy(
                    table_hbm.at[idx_v.at[pl.ds((cc * C0 + c) * RW, RW)]],
                    rows_v.at[b, pl.ds(c * RW, RW)],
                    gsem.at[b],
                )

        def wait_gathers(b):
            pltpu.make_async_copy(
                table_hbm.at[pl.ds(0, C0 * RW)], rows_v.at[b], gsem.at[b]
            ).wait()

        def fire_out(cc, wdg):
            for c in range(C0):
                pltpu.async_copy(
                    obuf_v.at[c],
                    out_hbm.at[
                        cc * C0 + c,
                        pl.ds(0, EMBED_DIM),
                        pl.ds(wdg * RW, RW),
                    ],
                    osem,
                )

        def wait_out():
            pltpu.make_async_copy(
                out_hbm.at[pl.ds(0, C0), pl.ds(0, EMBED_DIM), pl.ds(0, RW)],
                obuf_v,
                osem,
            ).wait()

        cvs = [jnp.full((16,), c, jnp.int32) for c in range(C0)]
        evs = [iota16 + h * 16 for h in range(EMBED_DIM // 16)]

        def transpose_store(b):
            for c in range(C0):
                @plsc.parallel_loop(0, RW, unroll=16)
                def _(r):
                    colv = jnp.full((16,), r, jnp.int32)
                    for h in range(EMBED_DIM // 16):
                        v = rows_v[b, c * RW + r, pl.ds(h * 16, 16)]
                        plsc.store_scatter(obuf_v, [cvs[c], evs[h], colv], v)

        def window(k, carry):
            wdg = wid * NWIN + k
            pltpu.async_copy(idx_hbm.at[pl.ds(wdg * slab, slab)], idx_v, isem)
            pltpu.make_async_copy(
                idx_hbm.at[pl.ds(0, slab)], idx_v, isem
            ).wait()
            fire_gathers(0, 0)
            for cc in range(NCHUNK):
                b = cc % 2
                if cc + 1 < NCHUNK:
                    fire_gathers(cc + 1, 1 - b)
                wait_gathers(b)
                if cc == 0:
                    @pl.when(k > 0)
                    def _():
                        wait_out()
                else:
                    wait_out()
                transpose_store(b)
                fire_out(cc, wdg)
            return carry

        lax.fori_loop(0, NWIN, window, 0)
        wait_out()

    return emb


def kernel(x, table):
    num_rows, num_cols = x.shape
    vocab = table.shape[0]
    n_full = vocab // RW
    vocab_pad = (n_full + 1) * RW

    xt = jnp.swapaxes(x, 0, 1).astype(jnp.int32)
    tt = jnp.swapaxes(table, 0, 1)
    tail = jnp.swapaxes(table[n_full * RW:], 0, 1)
    tailp = jnp.pad(tail, ((0, 0), (0, vocab_pad - vocab)))

    idx_lin, tbl_rm = _build_prep(num_rows, num_cols, vocab)(xt, tt, tailp)
    tbl_lin = tbl_rm.reshape(vocab_pad, EMBED_DIM)
    pout = _build_lookup(num_rows, num_cols, vocab_pad)(idx_lin, tbl_lin)
    return jnp.transpose(pout, (2, 0, 1))


# speedup vs baseline: 1.2192x; 1.2192x over previous
import functools
import math

import jax
import jax.numpy as jnp
from jax import lax
from jax.experimental import pallas as pl
from jax.experimental.pallas import tpu as pltpu
from jax.experimental.pallas import tpu_sc as plsc

EMBED_DIM = 32
SCALE = math.sqrt(EMBED_DIM)

NUM_CORES = 2
NUM_SUBCORES = 16
NUM_WORKERS = NUM_CORES * NUM_SUBCORES

RW = 128
NWIN = 4
C0 = 5
NCHUNK = 10


@functools.lru_cache(maxsize=None)
def _build_detile(num_rows: int, num_cols: int):
    n_tiles_c = (num_cols + 7) // 8
    n_win = num_rows // RW
    win_per_worker = n_win // NUM_WORKERS
    mesh = plsc.VectorSubcoreMesh(core_axis_name="c", subcore_axis_name="s")

    @functools.partial(
        pl.kernel,
        mesh=mesh,
        out_type=jax.ShapeDtypeStruct((num_rows * num_cols,), jnp.int32),
        scratch_types=[
            pltpu.VMEM((n_tiles_c * 8, RW), jnp.int32),
            pltpu.SemaphoreType.DMA,
            pltpu.SemaphoreType.DMA,
        ],
        compiler_params=pltpu.CompilerParams(
            use_tc_tiling_on_sc=True, needs_layout_passes=False
        ),
    )
    def detile(xt_hbm, out_hbm, stag, isem, osem):
        wid = lax.axis_index("s") * NUM_CORES + lax.axis_index("c")

        def win(k, carry):
            wdg = wid * win_per_worker + k
            r0 = wdg * RW
            for q in range(n_tiles_c):
                h = min(8, num_cols - q * 8)
                pltpu.async_copy(
                    xt_hbm.at[pl.ds(q * 8, h), pl.ds(r0, RW)],
                    stag.at[pl.ds(q * 8, h)],
                    isem,
                )
            for q in range(n_tiles_c):
                h = min(8, num_cols - q * 8)
                pltpu.make_async_copy(
                    xt_hbm.at[pl.ds(q * 8, h), pl.ds(r0, RW)],
                    stag.at[pl.ds(q * 8, h)],
                    isem,
                ).wait()
            for c in range(num_cols):
                pltpu.async_copy(
                    stag.at[c],
                    out_hbm.at[pl.ds((wdg * num_cols + c) * RW, RW)],
                    osem,
                )
            for c in range(num_cols):
                pltpu.make_async_copy(
                    stag.at[c],
                    out_hbm.at[pl.ds((wdg * num_cols + c) * RW, RW)],
                    osem,
                ).wait()
            return carry

        lax.fori_loop(0, win_per_worker, win, 0)

    return detile


@functools.lru_cache(maxsize=None)
def _build_lookup(num_rows: int, num_cols: int):
    assert num_rows == NUM_WORKERS * NWIN * RW
    assert num_cols == C0 * NCHUNK
    slab = num_cols * RW
    mesh = plsc.VectorSubcoreMesh(core_axis_name="c", subcore_axis_name="s")

    @functools.partial(
        pl.kernel,
        mesh=mesh,
        out_type=jax.ShapeDtypeStruct(
            (num_cols, EMBED_DIM, num_rows), jnp.float32
        ),
        scratch_types=[
            pltpu.VMEM((slab,), jnp.int32),
            pltpu.VMEM((2, C0 * RW, EMBED_DIM), jnp.float32),
            pltpu.VMEM((C0, EMBED_DIM, RW), jnp.float32),
            pltpu.SemaphoreType.DMA,
            pltpu.SemaphoreType.DMA((2,)),
            pltpu.SemaphoreType.DMA,
        ],
        compiler_params=pltpu.CompilerParams(
            use_tc_tiling_on_sc=False, needs_layout_passes=False
        ),
    )
    def emb(idx_hbm, table_hbm, out_hbm, idx_v, rows_v, obuf_v, isem, gsem, osem):
        wid = lax.axis_index("s") * NUM_CORES + lax.axis_index("c")
        iota16 = lax.iota(jnp.int32, 16)

        def fire_gathers(cc, b):
            for c in range(C0):
                pltpu.async_copy(
                    table_hbm.at[idx_v.at[pl.ds((cc * C0 + c) * RW, RW)]],
                    rows_v.at[b, pl.ds(c * RW, RW)],
                    gsem.at[b],
                )

        def wait_gathers(b):
            pltpu.make_async_copy(
                table_hbm.at[pl.ds(0, C0 * RW)], rows_v.at[b], gsem.at[b]
            ).wait()

        def fire_out(cc, wdg):
            for c in range(C0):
                pltpu.async_copy(
                    obuf_v.at[c],
                    out_hbm.at[
                        cc * C0 + c,
                        pl.ds(0, EMBED_DIM),
                        pl.ds(wdg * RW, RW),
                    ],
                    osem,
                )

        def wait_out():
            pltpu.make_async_copy(
                out_hbm.at[pl.ds(0, C0), pl.ds(0, EMBED_DIM), pl.ds(0, RW)],
                obuf_v,
                osem,
            ).wait()

        cvs = [jnp.full((16,), c, jnp.int32) for c in range(C0)]
        evs = [iota16 + h * 16 for h in range(EMBED_DIM // 16)]

        def transpose_scale(b):
            for c in range(C0):
                @plsc.parallel_loop(0, RW, unroll=16)
                def _(r):
                    colv = jnp.full((16,), r, jnp.int32)
                    for h in range(EMBED_DIM // 16):
                        v = rows_v[b, c * RW + r, pl.ds(h * 16, 16)] * SCALE
                        plsc.store_scatter(obuf_v, [cvs[c], evs[h], colv], v)

        def window(k, carry):
            wdg = wid * NWIN + k
            pltpu.async_copy(idx_hbm.at[pl.ds(wdg * slab, slab)], idx_v, isem)
            pltpu.make_async_copy(
                idx_hbm.at[pl.ds(0, slab)], idx_v, isem
            ).wait()
            fire_gathers(0, 0)
            for cc in range(NCHUNK):
                b = cc % 2
                if cc + 1 < NCHUNK:
                    fire_gathers(cc + 1, 1 - b)
                wait_gathers(b)
                if cc == 0:
                    @pl.when(k > 0)
                    def _():
                        wait_out()
                else:
                    wait_out()
                transpose_scale(b)
                fire_out(cc, wdg)
            return carry

        lax.fori_loop(0, NWIN, window, 0)
        wait_out()

    return emb


def kernel(x, table):
    num_rows, num_cols = x.shape
    xt = jnp.swapaxes(x, 0, 1).astype(jnp.int32)
    idx_lin = _build_detile(num_rows, num_cols)(xt)
    pout = _build_lookup(num_rows, num_cols)(idx_lin, table)
    return jnp.transpose(pout, (2, 0, 1))


# speedup vs baseline: 1.6736x; 1.3727x over previous
import functools
import math

import jax
import jax.numpy as jnp
from jax import lax
from jax.experimental import pallas as pl
from jax.experimental.pallas import tpu as pltpu
from jax.experimental.pallas import tpu_sc as plsc

EMBED_DIM = 32
SCALE = math.sqrt(EMBED_DIM)

NUM_CORES = 2
NUM_SUBCORES = 16
NUM_WORKERS = NUM_CORES * NUM_SUBCORES

RW = 128
NWIN = 4
C0 = 5
NCHUNK = 10


@functools.lru_cache(maxsize=None)
def _build_detile(num_rows: int, num_cols: int):
    n_tiles_c = (num_cols + 7) // 8
    n_win = num_rows // RW
    win_per_worker = n_win // NUM_WORKERS
    mesh = plsc.VectorSubcoreMesh(core_axis_name="c", subcore_axis_name="s")

    @functools.partial(
        pl.kernel,
        mesh=mesh,
        out_type=jax.ShapeDtypeStruct((num_rows * num_cols,), jnp.int32),
        scratch_types=[
            pltpu.VMEM((n_tiles_c * 8, RW), jnp.int32),
            pltpu.SemaphoreType.DMA,
            pltpu.SemaphoreType.DMA,
        ],
        compiler_params=pltpu.CompilerParams(
            use_tc_tiling_on_sc=True, needs_layout_passes=False
        ),
    )
    def detile(xt_hbm, out_hbm, stag, isem, osem):
        wid = lax.axis_index("s") * NUM_CORES + lax.axis_index("c")

        def win(k, carry):
            wdg = wid * win_per_worker + k
            r0 = wdg * RW
            for q in range(n_tiles_c):
                h = min(8, num_cols - q * 8)
                pltpu.async_copy(
                    xt_hbm.at[pl.ds(q * 8, h), pl.ds(r0, RW)],
                    stag.at[pl.ds(q * 8, h)],
                    isem,
                )
            for q in range(n_tiles_c):
                h = min(8, num_cols - q * 8)
                pltpu.make_async_copy(
                    xt_hbm.at[pl.ds(q * 8, h), pl.ds(r0, RW)],
                    stag.at[pl.ds(q * 8, h)],
                    isem,
                ).wait()
            for c in range(num_cols):
                pltpu.async_copy(
                    stag.at[c],
                    out_hbm.at[pl.ds((wdg * num_cols + c) * RW, RW)],
                    osem,
                )
            for c in range(num_cols):
                pltpu.make_async_copy(
                    stag.at[c],
                    out_hbm.at[pl.ds((wdg * num_cols + c) * RW, RW)],
                    osem,
                ).wait()
            return carry

        lax.fori_loop(0, win_per_worker, win, 0)

    return detile


@functools.lru_cache(maxsize=None)
def _build_lookup(num_rows: int, num_cols: int):
    assert num_rows == NUM_WORKERS * NWIN * RW
    assert num_cols == C0 * NCHUNK
    slab = num_cols * RW
    mesh = plsc.VectorSubcoreMesh(core_axis_name="c", subcore_axis_name="s")

    @functools.partial(
        pl.kernel,
        mesh=mesh,
        out_type=jax.ShapeDtypeStruct(
            (num_cols, EMBED_DIM, num_rows), jnp.float32
        ),
        scratch_types=[
            pltpu.VMEM((slab,), jnp.int32),
            pltpu.VMEM((2, C0 * RW, EMBED_DIM), jnp.float32),
            pltpu.VMEM((C0, EMBED_DIM, RW), jnp.float32),
            pltpu.SemaphoreType.DMA,
            pltpu.SemaphoreType.DMA((2,)),
            pltpu.SemaphoreType.DMA,
        ],
        compiler_params=pltpu.CompilerParams(
            use_tc_tiling_on_sc=False, needs_layout_passes=False
        ),
    )
    def emb(idx_hbm, table_hbm, out_hbm, idx_v, rows_v, obuf_v, isem, gsem, osem):
        wid = lax.axis_index("s") * NUM_CORES + lax.axis_index("c")
        iota16 = lax.iota(jnp.int32, 16)

        def fire_gathers(cc, b):
            for c in range(C0):
                pltpu.async_copy(
                    table_hbm.at[idx_v.at[pl.ds((cc * C0 + c) * RW, RW)]],
                    rows_v.at[b, pl.ds(c * RW, RW)],
                    gsem.at[b],
                )

        def wait_gathers(b):
            pltpu.make_async_copy(
                table_hbm.at[pl.ds(0, C0 * RW)], rows_v.at[b], gsem.at[b]
            ).wait()

        def fire_out(cc, wdg):
            for c in range(C0):
                pltpu.async_copy(
                    obuf_v.at[c],
                    out_hbm.at[
                        cc * C0 + c,
                        pl.ds(0, EMBED_DIM),
                        pl.ds(wdg * RW, RW),
                    ],
                    osem,
                )

        def wait_out():
            pltpu.make_async_copy(
                out_hbm.at[pl.ds(0, C0), pl.ds(0, EMBED_DIM), pl.ds(0, RW)],
                obuf_v,
                osem,
            ).wait()

        cvs = [jnp.full((16,), c, jnp.int32) for c in range(C0)]
        bvs = [jnp.full((16,), b, jnp.int32) for b in range(2)]

        def transpose_scale(b):
            for c in range(C0):
                @pl.loop(0, RW // 16)
                def _(rg):
                    ovec = iota16 + rg * 16
                    rvec = ovec + c * RW
                    for h in range(EMBED_DIM // 16):
                        @plsc.parallel_loop(0, 16, unroll=4)
                        def _(d):
                            evec = lax.bitwise_and(iota16 + d, 15) + h * 16
                            v = plsc.load_gather(
                                rows_v, [bvs[b], rvec, evec]
                            ) * SCALE
                            plsc.store_scatter(
                                obuf_v, [cvs[c], evec, ovec], v
                            )

        def window(k, carry):
            wdg = wid * NWIN + k
            pltpu.async_copy(idx_hbm.at[pl.ds(wdg * slab, slab)], idx_v, isem)
            pltpu.make_async_copy(
                idx_hbm.at[pl.ds(0, slab)], idx_v, isem
            ).wait()
            fire_gathers(0, 0)

            def cpair(s, carry2):
                for b in range(2):
                    cc = s * 2 + b
                    if b == 0:
                        fire_gathers(cc + 1, 1)
                    else:
                        @pl.when(s < NCHUNK // 2 - 1)
                        def _():
                            fire_gathers(cc + 1, 0)
                    wait_gathers(b)
                    if b == 0:
                        @pl.when((k > 0) | (s > 0))
                        def _():
                            wait_out()
                    else:
                        wait_out()
                    transpose_scale(b)
                    fire_out(cc, wdg)
                return carry2

            lax.fori_loop(0, NCHUNK // 2, cpair, 0)
            return carry

        lax.fori_loop(0, NWIN, window, 0)
        wait_out()

    return emb


def kernel(x, table):
    num_rows, num_cols = x.shape
    xt = jnp.swapaxes(x, 0, 1).astype(jnp.int32)
    idx_lin = _build_detile(num_rows, num_cols)(xt)
    pout = _build_lookup(num_rows, num_cols)(idx_lin, table)
    return jnp.transpose(pout, (2, 0, 1))
